# Initial kernel scaffold; baseline (speedup 1.0000x reference)
import jax
import jax.numpy as jnp
from jax.experimental import pallas as pl


def kernel(node_features, edge_index, num_nodes, num_edges, global_features, W_proj, b_proj, W_g1, b_g1, W_g2, b_g2, W_critic, b_critic, W_high, b_high, W_ltype, b_ltype, W_p1, b_p1, W_p2, b_p2, W_deploy, b_deploy, W_select, b_select):
    raise NotImplementedError("write your pallas kernel here")


# speedup vs baseline: 12.7078x; 12.7078x over previous
import functools

import jax
import jax.numpy as jnp
from jax.experimental import pallas as pl

B, N, E, F, H, S, L = 8, 2048, 32768, 256, 256, 2048, 8
RB = 256
NRB = N // RB


def _build_adj_jnp(edge_index):
    src = edge_index[:, 0, :]
    dst = edge_index[:, 1, :]
    g = jnp.arange(B, dtype=jnp.int32)[:, None]
    A = jnp.zeros((B, N, N), jnp.float32).at[g, dst, src].add(1.0)
    deg = jnp.sum(A, axis=2)
    return A, deg


def _encode_body(x_ref, deg_ref, wp_ref, bp_ref, w1_ref, u1_ref):
    x = x_ref[0]
    h0 = jnp.maximum(x @ wp_ref[...] + bp_ref[0], 0.0)
    dinv = jax.lax.rsqrt(deg_ref[0, 0] + 1.0)
    u1_ref[0] = (h0 * dinv[:, None]) @ w1_ref[...]


def _encode(x, deg, wp, bp, w1):
    return pl.pallas_call(
        _encode_body,
        grid=(B, NRB),
        in_specs=[
            pl.BlockSpec((1, RB, F), lambda g, r: (g, r, 0)),
            pl.BlockSpec((1, 1, RB), lambda g, r: (g, 0, r)),
            pl.BlockSpec((F, H), lambda g, r: (0, 0)),
            pl.BlockSpec((1, H), lambda g, r: (0, 0)),
            pl.BlockSpec((H, H), lambda g, r: (0, 0)),
        ],
        out_specs=pl.BlockSpec((1, RB, H), lambda g, r: (g, r, 0)),
        out_shape=jax.ShapeDtypeStruct((B, N, H), jnp.float32),
    )(x, deg, wp, bp, w1)


def _conv_proj_body(a_ref, u_ref, deg_ref, b_ref, wn_ref, out_ref):
    r = pl.program_id(1)
    u_full = u_ref[0]
    c = a_ref[0] @ u_full
    c = c + u_ref[0, pl.ds(r * RB, RB), :]
    dinv = jax.lax.rsqrt(deg_ref[0, 0, pl.ds(r * RB, RB)] + 1.0)
    h = jnp.maximum(c * dinv[:, None] + b_ref[0], 0.0)
    out_ref[0] = (h * dinv[:, None]) @ wn_ref[...]


def _conv_proj(A, u, deg, b, wn):
    return pl.pallas_call(
        _conv_proj_body,
        grid=(B, NRB),
        in_specs=[
            pl.BlockSpec((1, RB, N), lambda g, r: (g, r, 0)),
            pl.BlockSpec((1, N, H), lambda g, r: (g, 0, 0)),
            pl.BlockSpec((1, 1, N), lambda g, r: (g, 0, 0)),
            pl.BlockSpec((1, H), lambda g, r: (0, 0)),
            pl.BlockSpec((H, H), lambda g, r: (0, 0)),
        ],
        out_specs=pl.BlockSpec((1, RB, H), lambda g, r: (g, r, 0)),
        out_shape=jax.ShapeDtypeStruct((B, N, H), jnp.float32),
    )(A, u, deg, b, wn)


def _conv_pool_body(a_ref, u_ref, deg_ref, b_ref, ge_ref):
    r = pl.program_id(1)
    u_full = u_ref[0]
    c = a_ref[0] @ u_full
    c = c + u_ref[0, pl.ds(r * RB, RB), :]
    dinv = jax.lax.rsqrt(deg_ref[0, 0, pl.ds(r * RB, RB)] + 1.0)
    h = jnp.maximum(c * dinv[:, None] + b_ref[0], 0.0)
    part = jnp.sum(h, axis=0) * (1.0 / N)

    @pl.when(r == 0)
    def _():
        ge_ref[0, 0] = part

    @pl.when(r != 0)
    def _():
        ge_ref[0, 0] = ge_ref[0, 0] + part


def _conv_pool(A, u, deg, b):
    return pl.pallas_call(
        _conv_pool_body,
        grid=(B, NRB),
        in_specs=[
            pl.BlockSpec((1, RB, N), lambda g, r: (g, r, 0)),
            pl.BlockSpec((1, N, H), lambda g, r: (g, 0, 0)),
            pl.BlockSpec((1, 1, N), lambda g, r: (g, 0, 0)),
            pl.BlockSpec((1, H), lambda g, r: (0, 0)),
        ],
        out_specs=pl.BlockSpec((1, 1, H), lambda g, r: (g, 0, 0)),
        out_shape=jax.ShapeDtypeStruct((B, 1, H), jnp.float32),
    )(A, u, deg, b)


def _heads_body(ge_ref, wc_ref, bc_ref, wh_ref, bh_ref, wl_ref, bl_ref,
                wp1_ref, bp1_ref, wp2_ref, bp2_ref, wd_ref, bd_ref,
                ws_ref, bs_ref, o1, o2, o3, o4, o5, o6, o7):
    ge = ge_ref[...]
    o1[...] = ge @ wc_ref[...] + bc_ref[0]
    o2[...] = ge @ wh_ref[...] + bh_ref[0]
    o3[...] = ge @ wl_ref[...] + bl_ref[0]
    o4[...] = ge @ wp1_ref[...] + bp1_ref[0]
    o5[...] = ge @ wp2_ref[...] + bp2_ref[0]
    o6[...] = ge @ wd_ref[...] + bd_ref[0]
    o7[...] = ge @ ws_ref[...] + bs_ref[0]


def _heads(ge, wc, bc, wh, bh, wl, bl, wp1, bp1, wp2, bp2, wd, bd, ws, bs):
    full = lambda a: pl.BlockSpec(a.shape, lambda: tuple(0 for _ in a.shape))
    args = (ge, wc, bc, wh, bh, wl, bl, wp1, bp1, wp2, bp2, wd, bd, ws, bs)
    outs = [
        jax.ShapeDtypeStruct((B, 1), jnp.float32),
        jax.ShapeDtypeStruct((B, 4), jnp.float32),
        jax.ShapeDtypeStruct((B, 3), jnp.float32),
        jax.ShapeDtypeStruct((B, S), jnp.float32),
        jax.ShapeDtypeStruct((B, S), jnp.float32),
        jax.ShapeDtypeStruct((B, S), jnp.float32),
        jax.ShapeDtypeStruct((B, L), jnp.float32),
    ]
    return pl.pallas_call(
        _heads_body,
        in_specs=[full(a) for a in args],
        out_specs=[pl.BlockSpec(o.shape, lambda: tuple(0 for _ in o.shape))
                   for o in outs],
        out_shape=outs,
    )(*args)


def kernel(node_features, edge_index, num_nodes, num_edges, global_features,
           W_proj, b_proj, W_g1, b_g1, W_g2, b_g2, W_critic, b_critic,
           W_high, b_high, W_ltype, b_ltype, W_p1, b_p1, W_p2, b_p2,
           W_deploy, b_deploy, W_select, b_select):
    A, deg = _build_adj_jnp(edge_index)
    deg3 = deg.reshape(B, 1, N)
    r2 = lambda v: v.reshape(1, -1)
    u1 = _encode(node_features, deg3, W_proj, r2(b_proj), W_g1)
    u2 = _conv_proj(A, u1, deg3, r2(b_g1), W_g2)
    ge = _conv_pool(A, u2, deg3, r2(b_g2)).reshape(B, H)
    return _heads(ge, W_critic, r2(b_critic), W_high, r2(b_high),
                  W_ltype, r2(b_ltype), W_p1, r2(b_p1), W_p2, r2(b_p2),
                  W_deploy, r2(b_deploy), W_select, r2(b_select))
